# baseline (speedup 1.0000x reference)
import functools

import jax
import jax.numpy as jnp
from jax import lax
from jax.experimental import pallas as pl
from jax.experimental.pallas import tpu as pltpu
from jax.experimental.pallas import tpu_sc as plsc

LANES = 16
PADE = 128
RC = 200


def kernel(inputs, token_table, pos_table):
    B, T = inputs.shape
    V, E = token_table.shape

    info = plsc.get_sparse_core_info()
    nc, ns = info.num_cores, info.num_subcores
    nw = nc * ns

    n_relay = V // RC
    relay_per_w = n_relay // nw
    n_extra = n_relay - relay_per_w * nw
    relay_groups = relay_per_w // 2
    assert V % RC == 0 and relay_per_w % 2 == 0 and RC % 8 == 0

    seq_per_w = B // nw
    n_groups = seq_per_w // 2
    assert B % (2 * nw) == 0 and E % LANES == 0 and T % 8 == 0

    mesh = plsc.VectorSubcoreMesh(core_axis_name="c", subcore_axis_name="s")
    params = pltpu.CompilerParams(use_tc_tiling_on_sc=True)

    @functools.partial(
        pl.kernel,
        mesh=mesh,
        out_type=jax.ShapeDtypeStruct((V, PADE), jnp.float32),
        scratch_types=[
            [pltpu.VMEM((RC, E), jnp.float32) for _ in range(2)],
            [pltpu.VMEM((RC, PADE), jnp.float32) for _ in range(2)],
            [pltpu.SemaphoreType.DMA for _ in range(2)],
        ],
        compiler_params=params,
    )
    def relayout(table_hbm, pad_hbm, nar, wide, sem_w):
        wid = lax.axis_index("s") * nc + lax.axis_index("c")

        def widen(b):
            @plsc.parallel_loop(0, RC, unroll=8)
            def _(r):
                for k in range(E // LANES):
                    sl = pl.ds(k * LANES, LANES)
                    wide[b][r, sl] = nar[b][r, sl]

        def move(j, b, first):
            row0 = (wid + j * nw) * RC
            pltpu.sync_copy(table_hbm.at[pl.ds(row0, RC)], nar[b])
            if not first:
                pltpu.make_async_copy(
                    wide[b], pad_hbm.at[pl.ds(0, RC)], sem_w[b]).wait()
            widen(b)
            pltpu.async_copy(wide[b], pad_hbm.at[pl.ds(row0, RC)], sem_w[b])

        move(0, 0, True)
        move(1, 1, True)

        def group(g, carry):
            for b in range(2):
                @pl.when(g <= relay_groups - 2)
                def _():
                    move((g + 1) * 2 + b, b, False)
            return carry

        lax.fori_loop(0, relay_groups - 1, group, 0)
        for b in range(2):
            pltpu.make_async_copy(
                wide[b], pad_hbm.at[pl.ds(0, RC)], sem_w[b]).wait()

        @pl.when(wid < n_extra)
        def _():
            row0 = (relay_per_w * nw + wid) * RC
            pltpu.sync_copy(table_hbm.at[pl.ds(row0, RC)], nar[0])
            widen(0)
            pltpu.sync_copy(wide[0], pad_hbm.at[pl.ds(row0, RC)])

    @functools.partial(
        pl.kernel,
        mesh=mesh,
        out_type=jax.ShapeDtypeStruct((B, T, E), jnp.float32),
        scratch_types=[
            [pltpu.VMEM((T,), jnp.int32) for _ in range(2)],
            [pltpu.VMEM((T, PADE), jnp.float32) for _ in range(2)],
            [pltpu.VMEM((T, E), jnp.float32) for _ in range(2)],
            pltpu.VMEM((T, E), jnp.float32),
            [pltpu.SemaphoreType.DMA for _ in range(2)],
            [pltpu.SemaphoreType.DMA for _ in range(2)],
        ],
        compiler_params=params,
    )
    def lookup(idx_hbm, pad_hbm, pos_hbm, out_hbm, idx_c, bufs, obufs,
               pos_v, sem_g, sem_st):
        wid = lax.axis_index("s") * nc + lax.axis_index("c")
        base = wid * seq_per_w
        pltpu.sync_copy(pos_hbm, pos_v)

        def start_gather(c, b):
            pltpu.sync_copy(idx_hbm.at[pl.ds((base + c) * T, T)], idx_c[b])
            pltpu.async_copy(pad_hbm.at[idx_c[b]], bufs[b], sem_g[b])

        def wait_gather(b):
            pltpu.make_async_copy(
                pad_hbm.at[pl.ds(0, T)], bufs[b], sem_g[b]).wait()

        def start_store(c, b):
            pltpu.async_copy(obufs[b], out_hbm.at[base + c], sem_st[b])

        def wait_store(b):
            pltpu.make_async_copy(obufs[b], out_hbm.at[0], sem_st[b]).wait()

        start_gather(0, 0)
        start_gather(1, 1)

        def group(g, carry):
            for b in range(2):
                c = g * 2 + b
                wait_gather(b)
                if b == 1:
                    @pl.when(g >= 1)
                    def _():
                        wait_store(1)

                @plsc.parallel_loop(0, T, unroll=8)
                def addrow(r):
                    for k in range(E // LANES):
                        sl = pl.ds(k * LANES, LANES)
                        obufs[b][r, sl] = bufs[b][r, sl] + pos_v[r, sl]

                start_store(c, b)
                if b == 0:
                    @pl.when(g >= 1)
                    def _():
                        start_gather(g * 2 + 1, 1)
                else:
                    @pl.when(g <= n_groups - 2)
                    def _():
                        wait_store(0)
                        start_gather(g * 2 + 2, 0)
            return carry

        lax.fori_loop(0, n_groups, group, 0)
        wait_store(0)
        wait_store(1)

    pad_table = relayout(token_table.astype(jnp.float32))
    return lookup(inputs.reshape(-1).astype(jnp.int32), pad_table,
                  pos_table.astype(jnp.float32))


# speedup vs baseline: 1.2312x; 1.2312x over previous
import functools

import jax
import jax.numpy as jnp
from jax import lax
from jax.experimental import pallas as pl
from jax.experimental.pallas import tpu as pltpu
from jax.experimental.pallas import tpu_sc as plsc

LANES = 16
NBUF = 4


def kernel(inputs, token_table, pos_table):
    B, T = inputs.shape
    V, E = token_table.shape

    info = plsc.get_sparse_core_info()
    nc, ns = info.num_cores, info.num_subcores
    nw = nc * ns

    rows_total = B * T
    rows_per_w = rows_total // nw
    n_chunks = rows_per_w // T
    n_groups = n_chunks // NBUF
    assert B % (nw * NBUF) == 0 and E % LANES == 0 and T % 8 == 0

    idx = inputs.reshape(-1).astype(jnp.int32)

    mesh = plsc.VectorSubcoreMesh(core_axis_name="c", subcore_axis_name="s")

    @functools.partial(
        pl.kernel,
        mesh=mesh,
        out_type=jax.ShapeDtypeStruct((rows_total * E,), jnp.float32),
        scratch_types=[
            pltpu.VMEM((rows_per_w,), jnp.int32),
            [pltpu.VMEM((T, E), jnp.float32) for _ in range(NBUF)],
            [pltpu.VMEM((T * E,), jnp.float32) for _ in range(2)],
            pltpu.VMEM((T, E), jnp.float32),
            [pltpu.SemaphoreType.DMA for _ in range(NBUF)],
            [pltpu.SemaphoreType.DMA for _ in range(NBUF)],
        ],
        compiler_params=pltpu.CompilerParams(use_tc_tiling_on_sc=False),
    )
    def run(idx_hbm, tt_hbm, pos_hbm, out_hbm, idx_v, bufs, obufs, pos_v,
            sem_g, sem_st):
        wid = lax.axis_index("s") * nc + lax.axis_index("c")
        base = wid * rows_per_w
        pltpu.sync_copy(pos_hbm, pos_v)
        pltpu.sync_copy(idx_hbm.at[pl.ds(base, rows_per_w)], idx_v)

        def start_gather(c, b):
            pltpu.async_copy(
                tt_hbm.at[idx_v.at[pl.ds(c * T, T)]], bufs[b], sem_g[b])

        def wait_gather(b):
            pltpu.make_async_copy(
                tt_hbm.at[pl.ds(0, T)], bufs[b], sem_g[b]).wait()

        def start_store(c, o):
            pltpu.async_copy(
                obufs[o], out_hbm.at[pl.ds((base + c * T) * E, T * E)],
                sem_st[o])

        def wait_store(o):
            pltpu.make_async_copy(
                obufs[o], out_hbm.at[pl.ds(0, T * E)], sem_st[o]).wait()

        for b in range(NBUF):
            start_gather(b, b)

        def group(g, carry):
            for b in range(NBUF):
                c = g * NBUF + b
                o = b % 2
                wait_gather(b)
                if b >= 2:
                    wait_store(o)
                else:
                    @pl.when(g >= 1)
                    def _():
                        wait_store(o)

                @plsc.parallel_loop(0, T, unroll=8)
                def addrow(r):
                    for k in range(E // LANES):
                        sl = pl.ds(k * LANES, LANES)
                        obufs[o][pl.ds(r * E + k * LANES, LANES)] = (
                            bufs[b][r, sl] + pos_v[r, sl])

                start_store(c, o)
                if b == 0:
                    @pl.when(g >= 1)
                    def _():
                        start_gather(g * NBUF + NBUF - 1, NBUF - 1)
                else:
                    @pl.when(g <= n_groups - 2)
                    def _():
                        start_gather((g + 1) * NBUF + b - 1, b - 1)
            return carry

        lax.fori_loop(0, n_groups, group, 0)
        for o in range(2):
            wait_store(o)

    out = run(idx, token_table.astype(jnp.float32),
              pos_table.astype(jnp.float32))
    return out.reshape(B, T, E)
